# Initial kernel scaffold; baseline (speedup 1.0000x reference)
import jax
import jax.numpy as jnp
from jax.experimental import pallas as pl


def kernel(x, edge_index, batch, W1, b1, W2, b2, W3, b3, W4, b4, p1, p2, p3, p4, fcW, fcb, fc2W, fc2b):
    raise NotImplementedError("write your pallas kernel here")


# speedup vs baseline: 2.3700x; 2.3700x over previous
import functools
import jax
import jax.numpy as jnp
from jax.experimental import pallas as pl

_N = 100000
_E = 1600000
_G = 64
_NEG = jnp.float32(-3e38)
_BN = 2000


def _mm_body(x_ref, w_ref, o_ref):
    o_ref[...] = jnp.dot(x_ref[...], w_ref[...],
                         preferred_element_type=jnp.float32)


def _mm(x, W):
    n, din = x.shape
    dout = W.shape[1]
    grid = n // _BN
    return pl.pallas_call(
        _mm_body,
        grid=(grid,),
        in_specs=[
            pl.BlockSpec((_BN, din), lambda i: (i, 0)),
            pl.BlockSpec((din, dout), lambda i: (0, 0)),
        ],
        out_specs=pl.BlockSpec((_BN, dout), lambda i: (i, 0)),
        out_shape=jax.ShapeDtypeStruct((n, dout), jnp.float32),
    )(x, W)


def _combine_body(s0_ref, h_ref, esum_ref, k_ref, b_ref, p_ref, y_ref, sc_ref):
    K = k_ref[...]
    esum = esum_ref[...]
    deg = K * esum + K
    dinv = jnp.where(deg > 0, jax.lax.rsqrt(jnp.maximum(deg, 1e-12)), 0.0)
    u = dinv * K
    y = u * s0_ref[...] + (u * u) * h_ref[...] + b_ref[...]
    y = jnp.maximum(y, 0.0)
    y_ref[...] = y
    sc_ref[...] = jnp.tanh(jnp.sum(y * p_ref[...], axis=1, keepdims=True))


def _combine(S0, h, esum, K, b, p_hat):
    grid = _N // _BN
    f = h.shape[1]
    return pl.pallas_call(
        _combine_body,
        grid=(grid,),
        in_specs=[
            pl.BlockSpec((_BN, f), lambda i: (i, 0)),
            pl.BlockSpec((_BN, f), lambda i: (i, 0)),
            pl.BlockSpec((_BN, 1), lambda i: (i, 0)),
            pl.BlockSpec((_BN, 1), lambda i: (i, 0)),
            pl.BlockSpec((1, f), lambda i: (0, 0)),
            pl.BlockSpec((1, f), lambda i: (0, 0)),
        ],
        out_specs=[
            pl.BlockSpec((_BN, f), lambda i: (i, 0)),
            pl.BlockSpec((_BN, 1), lambda i: (i, 0)),
        ],
        out_shape=[
            jax.ShapeDtypeStruct((_N, f), jnp.float32),
            jax.ShapeDtypeStruct((_N, 1), jnp.float32),
        ],
    )(S0, h, esum.reshape(_N, 1), K.reshape(_N, 1),
      b.reshape(1, f), p_hat.reshape(1, f))


def _mlp_body(xc_ref, w1_ref, b1_ref, w2_ref, b2_ref, o_ref):
    hh = jnp.maximum(
        jnp.dot(xc_ref[...], w1_ref[...], preferred_element_type=jnp.float32)
        + b1_ref[...], 0.0)
    o_ref[...] = (jnp.dot(hh, w2_ref[...], preferred_element_type=jnp.float32)
                  + b2_ref[...])


def _mlp(x_c, fcW, fcb, fc2W, fc2b):
    g = x_c.shape[0]
    return pl.pallas_call(
        _mlp_body,
        out_shape=jax.ShapeDtypeStruct((g, 1), jnp.float32),
    )(x_c, fcW, fcb.reshape(1, -1), fc2W, fc2b.reshape(1, 1))


@jax.jit
def _forward_impl(x, edge_index, batch, W1, b1, W2, b2, W3, b3, W4, b4,
                  p1, p2, p3, p4, fcW, fcb, fc2W, fc2b):
    src = edge_index[0]
    dst = edge_index[1]
    seg = batch
    n_per = jax.ops.segment_sum(jnp.ones((_N,), jnp.int32), seg,
                                num_segments=_G)
    starts = jnp.concatenate(
        [jnp.zeros((1,), jnp.int32),
         jnp.cumsum(n_per)[:-1].astype(jnp.int32)])
    K = jnp.ones((_N,), jnp.float32)
    pos = jnp.arange(_N, dtype=jnp.int32)

    for (W, b, p) in ((W1, b1, p1), (W2, b2, p2), (W3, b3, p3), (W4, b4, p4)):
        h = _mm(x, W)
        esum = jax.ops.segment_sum(K[src], dst, num_segments=_N)
        deg = K * esum + K
        dinv = jnp.where(deg > 0, jax.lax.rsqrt(jnp.maximum(deg, 1e-12)), 0.0)
        u = dinv * K
        g = u[:, None] * h
        S0 = jax.ops.segment_sum(g[src], dst, num_segments=_N)
        y, s2 = _combine(S0, h, esum, K, b, p / (jnp.linalg.norm(p) + 1e-16))
        s = s2[:, 0]
        s_eff = jnp.where(K > 0, s, _NEG)
        order = jnp.lexsort((-s_eff, seg))
        rank = jnp.zeros(_N, jnp.int32).at[order].set(pos - starts[seg[order]])
        cnt = jax.ops.segment_sum(K, seg, num_segments=_G)
        kk = jnp.ceil(0.5 * cnt)
        keep = (K > 0) & (rank.astype(jnp.float32) < kk[seg])
        K = keep.astype(jnp.float32)
        x = y * s[:, None] * K[:, None]

    x_for_max = jnp.where(K[:, None] > 0, x, _NEG)
    x_max = jax.ops.segment_max(x_for_max, seg, num_segments=_G)
    cnt = jax.ops.segment_sum(K, seg, num_segments=_G)
    x_sum = jax.ops.segment_sum(x * K[:, None], seg, num_segments=_G)
    x_mean = x_sum / jnp.maximum(cnt, 1.0)[:, None]
    x_c = jnp.concatenate([x_max, x_mean], axis=1)
    out = _mlp(x_c, fcW, fcb, fc2W, fc2b)
    return out[:, 0]


def kernel(x, edge_index, batch, W1, b1, W2, b2, W3, b3, W4, b4,
           p1, p2, p3, p4, fcW, fcb, fc2W, fc2b):
    return _forward_impl(x, edge_index, batch, W1, b1, W2, b2, W3, b3, W4, b4,
                         p1, p2, p3, p4, fcW, fcb, fc2W, fc2b)


# speedup vs baseline: 3.2387x; 1.3666x over previous
import functools
import jax
import jax.numpy as jnp
from jax import lax
from jax.experimental import pallas as pl
from jax.experimental.pallas import tpu as pltpu
from jax.experimental.pallas import tpu_sc as plsc

_N = 100000
_E = 1600000
_G = 64
_NEG = jnp.float32(-3e38)
_BN = 2000

_NSUB = 16
_EPS = _E // _NSUB
_CH = 4000
_RPS = 6256
_RPS_LAST = _N - 15 * _RPS


def _sc_msg_body(g0_hbm, g1_hbm, g2_hbm, g3_hbm, src_hbm, dst_hbm, zero_hbm,
                 out0_hbm, out1_hbm, out2_hbm, out3_hbm,
                 src_v, dst_v, rows_v, acc_sh, sem):
    c = lax.axis_index("c")
    s = lax.axis_index("s")

    def run(table_ref, out_ref):
        @pl.when(s < 15)
        def _():
            pltpu.sync_copy(zero_hbm.at[pl.ds(s * _RPS, _RPS)],
                            acc_sh.at[pl.ds(s * _RPS, _RPS)])

        @pl.when(s == 15)
        def _():
            pltpu.sync_copy(zero_hbm.at[pl.ds(15 * _RPS, _RPS_LAST)],
                            acc_sh.at[pl.ds(15 * _RPS, _RPS_LAST)])

        plsc.subcore_barrier()

        def body(i, _):
            off = s * _EPS + i * _CH
            pltpu.sync_copy(src_hbm.at[pl.ds(off, _CH)], src_v)
            pltpu.sync_copy(dst_hbm.at[pl.ds(off, _CH)], dst_v)
            pltpu.async_copy(table_ref.at[src_v], rows_v, sem).wait()
            pltpu.sync_copy(rows_v, acc_sh.at[dst_v], add=True)
            return ()

        lax.fori_loop(0, _EPS // _CH, body, ())
        plsc.subcore_barrier()

        @pl.when(s < 15)
        def _():
            pltpu.sync_copy(acc_sh.at[pl.ds(s * _RPS, _RPS)],
                            out_ref.at[pl.ds(s * _RPS, _RPS)])

        @pl.when(s == 15)
        def _():
            pltpu.sync_copy(acc_sh.at[pl.ds(15 * _RPS, _RPS_LAST)],
                            out_ref.at[pl.ds(15 * _RPS, _RPS_LAST)])

        plsc.subcore_barrier()

    @pl.when(c == 0)
    def _():
        run(g0_hbm, out0_hbm)
        run(g1_hbm, out1_hbm)

    @pl.when(c == 1)
    def _():
        run(g2_hbm, out2_hbm)
        run(g3_hbm, out3_hbm)


_OUT8 = jax.ShapeDtypeStruct((_N, 8), jnp.float32)


@functools.partial(
    pl.kernel,
    mesh=plsc.VectorSubcoreMesh(core_axis_name="c", subcore_axis_name="s"),
    compiler_params=pltpu.CompilerParams(use_tc_tiling_on_sc=False),
    out_type=[_OUT8, _OUT8, _OUT8, _OUT8],
    scratch_types=[
        pltpu.VMEM((_CH,), jnp.int32),
        pltpu.VMEM((_CH,), jnp.int32),
        pltpu.VMEM((_CH, 8), jnp.float32),
        pltpu.VMEM_SHARED((_N, 8), jnp.float32),
        pltpu.SemaphoreType.DMA,
    ],
)
def _sc_msg(g0_hbm, g1_hbm, g2_hbm, g3_hbm, src_hbm, dst_hbm, zero_hbm,
            out0_hbm, out1_hbm, out2_hbm, out3_hbm,
            src_v, dst_v, rows_v, acc_sh, sem):
    _sc_msg_body(g0_hbm, g1_hbm, g2_hbm, g3_hbm, src_hbm, dst_hbm, zero_hbm,
                 out0_hbm, out1_hbm, out2_hbm, out3_hbm,
                 src_v, dst_v, rows_v, acc_sh, sem)


def _mm_body(x_ref, w_ref, o_ref):
    o_ref[...] = jnp.dot(x_ref[...], w_ref[...],
                         preferred_element_type=jnp.float32)


def _mm(x, W):
    n, din = x.shape
    dout = W.shape[1]
    grid = n // _BN
    return pl.pallas_call(
        _mm_body,
        grid=(grid,),
        in_specs=[
            pl.BlockSpec((_BN, din), lambda i: (i, 0)),
            pl.BlockSpec((din, dout), lambda i: (0, 0)),
        ],
        out_specs=pl.BlockSpec((_BN, dout), lambda i: (i, 0)),
        out_shape=jax.ShapeDtypeStruct((n, dout), jnp.float32),
    )(x, W)


def _combine_body(s0_ref, h_ref, esum_ref, k_ref, b_ref, p_ref, y_ref, sc_ref):
    K = k_ref[...]
    esum = esum_ref[...]
    deg = K * esum + K
    dinv = jnp.where(deg > 0, jax.lax.rsqrt(jnp.maximum(deg, 1e-12)), 0.0)
    u = dinv * K
    y = u * s0_ref[...] + (u * u) * h_ref[...] + b_ref[...]
    y = jnp.maximum(y, 0.0)
    y_ref[...] = y
    sc_ref[...] = jnp.tanh(jnp.sum(y * p_ref[...], axis=1, keepdims=True))


def _combine(S0, h, esum, K, b, p_hat):
    grid = _N // _BN
    f = h.shape[1]
    return pl.pallas_call(
        _combine_body,
        grid=(grid,),
        in_specs=[
            pl.BlockSpec((_BN, f), lambda i: (i, 0)),
            pl.BlockSpec((_BN, f), lambda i: (i, 0)),
            pl.BlockSpec((_BN, 1), lambda i: (i, 0)),
            pl.BlockSpec((_BN, 1), lambda i: (i, 0)),
            pl.BlockSpec((1, f), lambda i: (0, 0)),
            pl.BlockSpec((1, f), lambda i: (0, 0)),
        ],
        out_specs=[
            pl.BlockSpec((_BN, f), lambda i: (i, 0)),
            pl.BlockSpec((_BN, 1), lambda i: (i, 0)),
        ],
        out_shape=[
            jax.ShapeDtypeStruct((_N, f), jnp.float32),
            jax.ShapeDtypeStruct((_N, 1), jnp.float32),
        ],
    )(S0, h, esum.reshape(_N, 1), K.reshape(_N, 1),
      b.reshape(1, f), p_hat.reshape(1, f))


def _mlp_body(xc_ref, w1_ref, b1_ref, w2_ref, b2_ref, o_ref):
    hh = jnp.maximum(
        jnp.dot(xc_ref[...], w1_ref[...], preferred_element_type=jnp.float32)
        + b1_ref[...], 0.0)
    o_ref[...] = (jnp.dot(hh, w2_ref[...], preferred_element_type=jnp.float32)
                  + b2_ref[...])


def _mlp(x_c, fcW, fcb, fc2W, fc2b):
    g = x_c.shape[0]
    return pl.pallas_call(
        _mlp_body,
        out_shape=jax.ShapeDtypeStruct((g, 1), jnp.float32),
    )(x_c, fcW, fcb.reshape(1, -1), fc2W, fc2b.reshape(1, 1))


@jax.jit
def _forward_impl(x, edge_index, batch, W1, b1, W2, b2, W3, b3, W4, b4,
                  p1, p2, p3, p4, fcW, fcb, fc2W, fc2b):
    src = edge_index[0]
    dst = edge_index[1]
    seg = batch
    n_per = jax.ops.segment_sum(jnp.ones((_N,), jnp.int32), seg,
                                num_segments=_G)
    starts = jnp.concatenate(
        [jnp.zeros((1,), jnp.int32),
         jnp.cumsum(n_per)[:-1].astype(jnp.int32)])
    K = jnp.ones((_N,), jnp.float32)
    pos = jnp.arange(_N, dtype=jnp.int32)

    for (W, b, p) in ((W1, b1, p1), (W2, b2, p2), (W3, b3, p3), (W4, b4, p4)):
        h = _mm(x, W)
        esum = jax.ops.segment_sum(K[src], dst, num_segments=_N)
        deg = K * esum + K
        dinv = jnp.where(deg > 0, jax.lax.rsqrt(jnp.maximum(deg, 1e-12)), 0.0)
        u = dinv * K
        g = u[:, None] * h
        outs = _sc_msg(g[:, :8], g[:, 8:16], g[:, 16:24], g[:, 24:],
                       src, dst, jnp.zeros((_N, 8), jnp.float32))
        S0 = jnp.concatenate(outs, axis=1)
        y, s2 = _combine(S0, h, esum, K, b, p / (jnp.linalg.norm(p) + 1e-16))
        s = s2[:, 0]
        s_eff = jnp.where(K > 0, s, _NEG)
        order = jnp.lexsort((-s_eff, seg))
        rank = jnp.zeros(_N, jnp.int32).at[order].set(pos - starts[seg[order]])
        cnt = jax.ops.segment_sum(K, seg, num_segments=_G)
        kk = jnp.ceil(0.5 * cnt)
        keep = (K > 0) & (rank.astype(jnp.float32) < kk[seg])
        K = keep.astype(jnp.float32)
        x = y * s[:, None] * K[:, None]

    x_for_max = jnp.where(K[:, None] > 0, x, _NEG)
    x_max = jax.ops.segment_max(x_for_max, seg, num_segments=_G)
    cnt = jax.ops.segment_sum(K, seg, num_segments=_G)
    x_sum = jax.ops.segment_sum(x * K[:, None], seg, num_segments=_G)
    x_mean = x_sum / jnp.maximum(cnt, 1.0)[:, None]
    x_c = jnp.concatenate([x_max, x_mean], axis=1)
    out = _mlp(x_c, fcW, fcb, fc2W, fc2b)
    return out[:, 0]


def kernel(x, edge_index, batch, W1, b1, W2, b2, W3, b3, W4, b4,
           p1, p2, p3, p4, fcW, fcb, fc2W, fc2b):
    return _forward_impl(x, edge_index, batch, W1, b1, W2, b2, W3, b3, W4, b4,
                         p1, p2, p3, p4, fcW, fcb, fc2W, fc2b)
